# Initial kernel scaffold; baseline (speedup 1.0000x reference)
import jax
import jax.numpy as jnp
from jax.experimental import pallas as pl


def kernel(x, edge_index, W, att_src, att_dst, bias):
    raise NotImplementedError("write your pallas kernel here")


# speedup vs baseline: 21.2955x; 21.2955x over previous
import dataclasses

import jax
import jax.numpy as jnp
from jax import lax
from jax.experimental import pallas as pl
from jax.experimental.pallas import tpu as pltpu
from jax.experimental.pallas import tpu_sc as plsc

N = 10000
E = 320000
D = 128
C = 128

NC = 2
NS = 16
NT = NC * NS
CHUNK = 64
NCHUNK = 160
EPTP = NCHUNK * CHUNK
E_PAD = NT * EPTP
NBUF = 4
NZC = N // CHUNK
NREM = N - NZC * CHUNK
MMB = 1000


def _mm_body(x_ref, w_ref, asv_ref, adv_ref,
             h_ref, as_ref, ad_ref, ms_ref, md_ref):
    i = pl.program_id(0)
    h = jnp.dot(x_ref[...], w_ref[...], preferred_element_type=jnp.float32)
    h_ref[...] = h
    a_s = jnp.sum(h * asv_ref[...], axis=1, keepdims=True)
    a_d = jnp.sum(h * adv_ref[...], axis=1, keepdims=True)
    as_ref[...] = a_s
    ad_ref[...] = a_d

    @pl.when(i == 0)
    def _():
        ms_ref[...] = jnp.full((1, 128), -1e30, jnp.float32)
        md_ref[...] = jnp.full((1, 128), -1e30, jnp.float32)

    ms_ref[...] = jnp.maximum(ms_ref[...], jnp.max(a_s))
    md_ref[...] = jnp.maximum(md_ref[...], jnp.max(a_d))


_mm = pl.pallas_call(
    _mm_body,
    grid=(N // MMB,),
    in_specs=[
        pl.BlockSpec((MMB, D), lambda i: (i, 0)),
        pl.BlockSpec((D, C), lambda i: (0, 0)),
        pl.BlockSpec((1, C), lambda i: (0, 0)),
        pl.BlockSpec((1, C), lambda i: (0, 0)),
    ],
    out_specs=[
        pl.BlockSpec((MMB, C), lambda i: (i, 0)),
        pl.BlockSpec((MMB, 1), lambda i: (i, 0)),
        pl.BlockSpec((MMB, 1), lambda i: (i, 0)),
        pl.BlockSpec((1, C), lambda i: (0, 0)),
        pl.BlockSpec((1, C), lambda i: (0, 0)),
    ],
    out_shape=[
        jax.ShapeDtypeStruct((N, C), jnp.float32),
        jax.ShapeDtypeStruct((N, 1), jnp.float32),
        jax.ShapeDtypeStruct((N, 1), jnp.float32),
        jax.ShapeDtypeStruct((1, C), jnp.float32),
        jax.ShapeDtypeStruct((1, C), jnp.float32),
    ],
)


def _sc_body(h_hbm, asrc_hbm, adst_hbm, src_hbm, dst_hbm, mv_hbm,
             o_hbm, s0_hbm, s1_hbm,
             sidx, didx, asg, adg, exb, rows, sz, m_v,
             out_sh, s_sh,
             i0, i1, i2, i3, g0, g1, g2, g3, c0, c1, c2, c3):
    core = lax.axis_index("c")
    sub = lax.axis_index("s")
    isems = [i0, i1, i2, i3]
    gsems = [g0, g1, g2, g3]
    ssems = [c0, c1, c2, c3]

    zv = jnp.zeros((16,), jnp.float32)

    @pl.loop(0, CHUNK)
    def _(r):
        for g in range(8):
            rows[0, r, pl.ds(g * 16, 16)] = zv

    @pl.loop(0, 1024, step=16)
    def _(i):
        sz[pl.ds(i, 16)] = zv

    for k in range(10):
        cid = sub + 16 * k

        @pl.when(cid < NZC)
        def _():
            pltpu.sync_copy(rows.at[0], out_sh.at[pl.ds(cid * CHUNK, CHUNK)])

    @pl.when(sub == 1)
    def _():
        pltpu.sync_copy(rows.at[0].at[pl.ds(0, NREM)],
                        out_sh.at[pl.ds(NZC * CHUNK, NREM)])

    @pl.when(sub == 0)
    def _():
        for i in range(10):
            pltpu.sync_copy(sz.at[pl.ds(0, 1000)],
                            s_sh.at[pl.ds(i * 1000, 1000)])

    pltpu.sync_copy(mv_hbm, m_v)
    plsc.subcore_barrier()

    mv = m_v[...]
    w = core * NS + sub
    ebase0 = w * EPTP

    def idx_start(j, b):
        pltpu.async_copy(src_hbm.at[pl.ds(ebase0 + j * CHUNK, CHUNK)],
                         sidx.at[b], isems[b])
        pltpu.async_copy(dst_hbm.at[pl.ds(ebase0 + j * CHUNK, CHUNK)],
                         didx.at[b], isems[b])

    def idx_wait(j, b):
        pltpu.make_async_copy(src_hbm.at[pl.ds(ebase0 + j * CHUNK, CHUNK)],
                              sidx.at[b], isems[b]).wait()
        pltpu.make_async_copy(dst_hbm.at[pl.ds(ebase0 + j * CHUNK, CHUNK)],
                              didx.at[b], isems[b]).wait()

    def gathers_start(b):
        pltpu.async_copy(h_hbm.at[sidx.at[b]], rows.at[b], gsems[b])
        pltpu.async_copy(asrc_hbm.at[sidx.at[b]], asg.at[b], gsems[b])
        pltpu.async_copy(adst_hbm.at[didx.at[b]], adg.at[b], gsems[b])

    def gathers_wait(b):
        pltpu.make_async_copy(h_hbm.at[sidx.at[b]], rows.at[b],
                              gsems[b]).wait()
        pltpu.make_async_copy(asrc_hbm.at[sidx.at[b]], asg.at[b],
                              gsems[b]).wait()
        pltpu.make_async_copy(adst_hbm.at[didx.at[b]], adg.at[b],
                              gsems[b]).wait()

    def scatter_start(b):
        pltpu.async_copy(rows.at[b], out_sh.at[didx.at[b]], ssems[b],
                         add=True)

    def scatter_wait(b):
        pltpu.make_async_copy(rows.at[b], out_sh.at[didx.at[b]],
                              ssems[b]).wait()

    pltpu.sync_copy(src_hbm.at[pl.ds(ebase0, CHUNK)], sidx.at[0])
    pltpu.sync_copy(dst_hbm.at[pl.ds(ebase0, CHUNK)], didx.at[0])
    gathers_start(0)
    idx_start(1, 1)

    @pl.loop(0, NCHUNK // NBUF)
    def _(o):
        for b in range(NBUF):
            j = o * NBUF + b
            b1 = (b + 1) % NBUF
            b2 = (b + 2) % NBUF
            last = (b == NBUF - 1)

            def _prep():
                idx_wait(j + 1, b1)

            def _gath():
                gathers_start(b1)

            if last:
                @pl.when(o < NCHUNK // NBUF - 1)
                def _():
                    _prep()
            else:
                _prep()

            if b < 2:
                @pl.when(o > 0)
                def _():
                    scatter_wait(b2)
            else:
                scatter_wait(b2)

            if last:
                @pl.when(o < NCHUNK // NBUF - 1)
                def _():
                    _gath()
            else:
                _gath()

            gathers_wait(b)

            def _pref2():
                idx_start(j + 2, b2)

            if b >= NBUF - 2:
                @pl.when(j + 2 < NCHUNK)
                def _():
                    _pref2()
            else:
                _pref2()

            for k in range(CHUNK // 16):
                e = (asg[b, pl.ds(k * 16, 16)] + adg[b, pl.ds(k * 16, 16)])
                e = jnp.where(e > 0, e, e * 0.2)
                exv = jnp.exp(e - mv)
                gid = (ebase0 + j * CHUNK + k * 16
                       + lax.iota(jnp.int32, 16))
                exb[pl.ds(b * CHUNK + k * 16, 16)] = jnp.where(
                    gid < E, exv, 0.0)

            pltpu.sync_copy(exb.at[pl.ds(b * CHUNK, CHUNK)],
                            s_sh.at[didx.at[b]], add=True)

            @pl.loop(0, CHUNK)
            def _(e):
                eidx = jnp.full((16,), b * CHUNK + e, jnp.int32)
                avec = plsc.load_gather(exb, [eidx])
                for g in range(8):
                    rows[b, e, pl.ds(g * 16, 16)] = (
                        rows[b, e, pl.ds(g * 16, 16)] * avec)

            scatter_start(b)

    scatter_wait(NBUF - 2)
    scatter_wait(NBUF - 1)
    plsc.subcore_barrier()

    for k in range(10):
        cid = sub + 16 * k

        @pl.when(cid < NZC)
        def _():
            pltpu.sync_copy(out_sh.at[pl.ds(cid * CHUNK, CHUNK)],
                            o_hbm.at[core, pl.ds(cid * CHUNK, CHUNK)])

    @pl.when(sub == 1)
    def _():
        pltpu.sync_copy(out_sh.at[pl.ds(NZC * CHUNK, NREM)],
                        o_hbm.at[core, pl.ds(NZC * CHUNK, NREM)])

    @pl.when(sub == 0)
    def _():
        @pl.when(core == 0)
        def _():
            pltpu.sync_copy(s_sh, s0_hbm)

        @pl.when(core == 1)
        def _():
            pltpu.sync_copy(s_sh, s1_hbm)


_sc = pl.kernel(
    _sc_body,
    out_type=[
        jax.ShapeDtypeStruct((NC, N, C), jnp.float32),
        jax.ShapeDtypeStruct((N,), jnp.float32),
        jax.ShapeDtypeStruct((N,), jnp.float32),
    ],
    mesh=plsc.VectorSubcoreMesh(core_axis_name="c", subcore_axis_name="s"),
    scratch_types=[
        pltpu.VMEM((NBUF, CHUNK), jnp.int32),
        pltpu.VMEM((NBUF, CHUNK), jnp.int32),
        pltpu.VMEM((NBUF, CHUNK), jnp.float32),
        pltpu.VMEM((NBUF, CHUNK), jnp.float32),
        pltpu.VMEM((NBUF * CHUNK,), jnp.float32),
        pltpu.VMEM((NBUF, CHUNK, C), jnp.float32),
        pltpu.VMEM((1024,), jnp.float32),
        pltpu.VMEM((16,), jnp.float32),
        pltpu.VMEM_SHARED((N, C), jnp.float32),
        pltpu.VMEM_SHARED((N,), jnp.float32),
    ] + [pltpu.SemaphoreType.DMA] * 12,
    compiler_params=(
        dataclasses.replace(pltpu.CompilerParams(), needs_layout_passes=False)
        if "needs_layout_passes" in pltpu.CompilerParams.__dataclass_fields__
        else pltpu.CompilerParams()),
)


def _fin_body(o_ref, s0_ref, s1_ref, b_ref, out_ref):
    denom = s0_ref[...] + s1_ref[...] + 1e-16
    out_ref[...] = (o_ref[0] + o_ref[1]) / denom + b_ref[...]


_fin = pl.pallas_call(
    _fin_body,
    grid=(N // MMB,),
    in_specs=[
        pl.BlockSpec((NC, MMB, C), lambda i: (0, i, 0)),
        pl.BlockSpec((MMB, 1), lambda i: (i, 0)),
        pl.BlockSpec((MMB, 1), lambda i: (i, 0)),
        pl.BlockSpec((1, C), lambda i: (0, 0)),
    ],
    out_specs=pl.BlockSpec((MMB, C), lambda i: (i, 0)),
    out_shape=jax.ShapeDtypeStruct((N, C), jnp.float32),
)


@jax.jit
def kernel(x, edge_index, W, att_src, att_dst, bias):
    src = jnp.pad(edge_index[0], (0, E_PAD - E))
    dst = jnp.pad(edge_index[1], (0, E_PAD - E))
    h, a_s, a_d, ms, md = _mm(x, W, att_src.reshape(1, C),
                              att_dst.reshape(1, C))
    m0 = ms[0, 0] + md[0, 0]
    mstab = jnp.where(m0 > 0, m0, 0.2 * m0)
    mvec = jnp.full((16,), mstab, jnp.float32)
    o_part, s0, s1 = _sc(h, a_s.reshape(N), a_d.reshape(N), src, dst, mvec)
    return _fin(o_part, s0.reshape(N, 1), s1.reshape(N, 1), bias.reshape(1, C))


# speedup vs baseline: 46.1609x; 2.1676x over previous
import dataclasses

import jax
import jax.numpy as jnp
from jax import lax
from jax.experimental import pallas as pl
from jax.experimental.pallas import tpu as pltpu
from jax.experimental.pallas import tpu_sc as plsc

N = 10000
E = 320000
D = 128
C = 128

NC = 2
NS = 16
NT = NC * NS
CHUNK = 80
NCHUNK = 128
EPT = E // NT
EPTP = NCHUNK * CHUNK
E_PAD = NT * EPTP
NBUF = 4
NZC = N // CHUNK
MMB = 1000


def _mm_body(x_ref, w_ref, asv_ref, adv_ref,
             h_ref, as_ref, ad_ref, ms_ref, md_ref):
    i = pl.program_id(0)
    h = jnp.dot(x_ref[...], w_ref[...], preferred_element_type=jnp.float32)
    h_ref[...] = h
    a_s = jnp.sum(h * asv_ref[...], axis=1, keepdims=True)
    a_d = jnp.sum(h * adv_ref[...], axis=1, keepdims=True)
    as_ref[...] = a_s
    ad_ref[...] = a_d

    @pl.when(i == 0)
    def _():
        ms_ref[...] = jnp.full((1, 128), -1e30, jnp.float32)
        md_ref[...] = jnp.full((1, 128), -1e30, jnp.float32)

    ms_ref[...] = jnp.maximum(ms_ref[...], jnp.max(a_s))
    md_ref[...] = jnp.maximum(md_ref[...], jnp.max(a_d))


_mm = pl.pallas_call(
    _mm_body,
    grid=(N // MMB,),
    in_specs=[
        pl.BlockSpec((MMB, D), lambda i: (i, 0)),
        pl.BlockSpec((D, C), lambda i: (0, 0)),
        pl.BlockSpec((1, C), lambda i: (0, 0)),
        pl.BlockSpec((1, C), lambda i: (0, 0)),
    ],
    out_specs=[
        pl.BlockSpec((MMB, C), lambda i: (i, 0)),
        pl.BlockSpec((MMB, 1), lambda i: (i, 0)),
        pl.BlockSpec((MMB, 1), lambda i: (i, 0)),
        pl.BlockSpec((1, C), lambda i: (0, 0)),
        pl.BlockSpec((1, C), lambda i: (0, 0)),
    ],
    out_shape=[
        jax.ShapeDtypeStruct((N, C), jnp.float32),
        jax.ShapeDtypeStruct((N, 1), jnp.float32),
        jax.ShapeDtypeStruct((N, 1), jnp.float32),
        jax.ShapeDtypeStruct((1, C), jnp.float32),
        jax.ShapeDtypeStruct((1, C), jnp.float32),
    ],
)


def _sc_body(h_hbm, asrc_hbm, adst_hbm, src_hbm, dst_hbm, mv_hbm,
             o_hbm, s0_hbm, s1_hbm,
             sidx, didx, asg, adg, exb, rows, sz, m_v,
             out_sh, s_sh,
             i0, i1, i2, i3, g0, g1, g2, g3, c0, c1, c2, c3):
    core = lax.axis_index("c")
    sub = lax.axis_index("s")
    isems = [i0, i1, i2, i3]
    gsems = [g0, g1, g2, g3]
    ssems = [c0, c1, c2, c3]

    zv = jnp.zeros((16,), jnp.float32)

    @pl.loop(0, CHUNK)
    def _(r):
        for g in range(8):
            rows[0, r, pl.ds(g * 16, 16)] = zv

    @pl.loop(0, 1024, step=16)
    def _(i):
        sz[pl.ds(i, 16)] = zv

    for k in range(8):
        cid = sub + 16 * k

        @pl.when(cid < NZC)
        def _():
            pltpu.sync_copy(rows.at[0], out_sh.at[pl.ds(cid * CHUNK, CHUNK)])

    @pl.when(sub == 0)
    def _():
        for i in range(10):
            pltpu.sync_copy(sz.at[pl.ds(0, 1000)],
                            s_sh.at[pl.ds(i * 1000, 1000)])

    pltpu.sync_copy(mv_hbm, m_v)
    plsc.subcore_barrier()

    mv = m_v[...]
    w = core * NS + sub
    ebase0 = w * EPTP

    def idx_start(j, b):
        pltpu.async_copy(src_hbm.at[pl.ds(ebase0 + j * CHUNK, CHUNK)],
                         sidx.at[b], isems[b])
        pltpu.async_copy(dst_hbm.at[pl.ds(ebase0 + j * CHUNK, CHUNK)],
                         didx.at[b], isems[b])

    def idx_wait(j, b):
        pltpu.make_async_copy(src_hbm.at[pl.ds(ebase0 + j * CHUNK, CHUNK)],
                              sidx.at[b], isems[b]).wait()
        pltpu.make_async_copy(dst_hbm.at[pl.ds(ebase0 + j * CHUNK, CHUNK)],
                              didx.at[b], isems[b]).wait()

    def gathers_start(b):
        pltpu.async_copy(h_hbm.at[sidx.at[b]], rows.at[b], gsems[b])
        pltpu.async_copy(asrc_hbm.at[sidx.at[b]], asg.at[b], gsems[b])
        pltpu.async_copy(adst_hbm.at[didx.at[b]], adg.at[b], gsems[b])

    def gathers_wait(b):
        pltpu.make_async_copy(h_hbm.at[sidx.at[b]], rows.at[b],
                              gsems[b]).wait()
        pltpu.make_async_copy(asrc_hbm.at[sidx.at[b]], asg.at[b],
                              gsems[b]).wait()
        pltpu.make_async_copy(adst_hbm.at[didx.at[b]], adg.at[b],
                              gsems[b]).wait()

    def scatter_start(b):
        pltpu.async_copy(rows.at[b], out_sh.at[didx.at[b]], ssems[b],
                         add=True)

    def scatter_wait(b):
        pltpu.make_async_copy(rows.at[b], out_sh.at[didx.at[b]],
                              ssems[b]).wait()

    pltpu.sync_copy(src_hbm.at[pl.ds(ebase0, CHUNK)], sidx.at[0])
    pltpu.sync_copy(dst_hbm.at[pl.ds(ebase0, CHUNK)], didx.at[0])
    gathers_start(0)
    idx_start(1, 1)

    @pl.loop(0, NCHUNK // NBUF)
    def _(o):
        for b in range(NBUF):
            j = o * NBUF + b
            b1 = (b + 1) % NBUF
            b2 = (b + 2) % NBUF
            last = (b == NBUF - 1)

            def _prep():
                idx_wait(j + 1, b1)

            def _gath():
                gathers_start(b1)

            if last:
                @pl.when(o < NCHUNK // NBUF - 1)
                def _():
                    _prep()
            else:
                _prep()

            if b < 2:
                @pl.when(o > 0)
                def _():
                    scatter_wait(b2)
            else:
                scatter_wait(b2)

            if last:
                @pl.when(o < NCHUNK // NBUF - 1)
                def _():
                    _gath()
            else:
                _gath()

            gathers_wait(b)

            def _pref2():
                idx_start(j + 2, b2)

            if b >= NBUF - 2:
                @pl.when(j + 2 < NCHUNK)
                def _():
                    _pref2()
            else:
                _pref2()

            for k in range(CHUNK // 16):
                e = (asg[b, pl.ds(k * 16, 16)] + adg[b, pl.ds(k * 16, 16)])
                e = jnp.where(e > 0, e, e * 0.2)
                exv = jnp.exp(e - mv)
                lid = j * CHUNK + k * 16 + lax.iota(jnp.int32, 16)
                exb[pl.ds(b * CHUNK + k * 16, 16)] = jnp.where(
                    lid < EPT, exv, 0.0)

            pltpu.sync_copy(exb.at[pl.ds(b * CHUNK, CHUNK)],
                            s_sh.at[didx.at[b]], add=True)

            @pl.loop(0, CHUNK)
            def _(e):
                eidx = jnp.full((16,), b * CHUNK + e, jnp.int32)
                avec = plsc.load_gather(exb, [eidx])
                for g in range(8):
                    rows[b, e, pl.ds(g * 16, 16)] = (
                        rows[b, e, pl.ds(g * 16, 16)] * avec)

            scatter_start(b)

    scatter_wait(NBUF - 2)
    scatter_wait(NBUF - 1)
    plsc.subcore_barrier()

    for k in range(8):
        cid = sub + 16 * k

        @pl.when(cid < NZC)
        def _():
            pltpu.sync_copy(out_sh.at[pl.ds(cid * CHUNK, CHUNK)],
                            o_hbm.at[core, pl.ds(cid * CHUNK, CHUNK)])

    @pl.when(sub == 0)
    def _():
        @pl.when(core == 0)
        def _():
            pltpu.sync_copy(s_sh, s0_hbm)

        @pl.when(core == 1)
        def _():
            pltpu.sync_copy(s_sh, s1_hbm)


_sc = pl.kernel(
    _sc_body,
    out_type=[
        jax.ShapeDtypeStruct((NC, N, C), jnp.float32),
        jax.ShapeDtypeStruct((N,), jnp.float32),
        jax.ShapeDtypeStruct((N,), jnp.float32),
    ],
    mesh=plsc.VectorSubcoreMesh(core_axis_name="c", subcore_axis_name="s"),
    scratch_types=[
        pltpu.VMEM((NBUF, CHUNK), jnp.int32),
        pltpu.VMEM((NBUF, CHUNK), jnp.int32),
        pltpu.VMEM((NBUF, CHUNK), jnp.float32),
        pltpu.VMEM((NBUF, CHUNK), jnp.float32),
        pltpu.VMEM((NBUF * CHUNK,), jnp.float32),
        pltpu.VMEM((NBUF, CHUNK, C), jnp.float32),
        pltpu.VMEM((1024,), jnp.float32),
        pltpu.VMEM((16,), jnp.float32),
        pltpu.VMEM_SHARED((N, C), jnp.float32),
        pltpu.VMEM_SHARED((N,), jnp.float32),
    ] + [pltpu.SemaphoreType.DMA] * 12,
    compiler_params=(
        dataclasses.replace(pltpu.CompilerParams(), needs_layout_passes=False)
        if "needs_layout_passes" in pltpu.CompilerParams.__dataclass_fields__
        else pltpu.CompilerParams()),
)


def _fin_body(o_ref, s0_ref, s1_ref, b_ref, out_ref):
    denom = s0_ref[...] + s1_ref[...] + 1e-16
    out_ref[...] = (o_ref[0] + o_ref[1]) / denom + b_ref[...]


_fin = pl.pallas_call(
    _fin_body,
    grid=(N // MMB,),
    in_specs=[
        pl.BlockSpec((NC, MMB, C), lambda i: (0, i, 0)),
        pl.BlockSpec((MMB, 1), lambda i: (i, 0)),
        pl.BlockSpec((MMB, 1), lambda i: (i, 0)),
        pl.BlockSpec((1, C), lambda i: (0, 0)),
    ],
    out_specs=pl.BlockSpec((MMB, C), lambda i: (i, 0)),
    out_shape=jax.ShapeDtypeStruct((N, C), jnp.float32),
)


@jax.jit
def kernel(x, edge_index, W, att_src, att_dst, bias):
    pad = (jnp.arange(NT * (EPTP - EPT), dtype=edge_index.dtype) % N
           ).reshape(NT, EPTP - EPT)
    src = jnp.concatenate(
        [edge_index[0].reshape(NT, EPT), pad], axis=1).reshape(E_PAD)
    dst = jnp.concatenate(
        [edge_index[1].reshape(NT, EPT), pad], axis=1).reshape(E_PAD)
    h, a_s, a_d, ms, md = _mm(x, W, att_src.reshape(1, C),
                              att_dst.reshape(1, C))
    m0 = ms[0, 0] + md[0, 0]
    mstab = jnp.where(m0 > 0, m0, 0.2 * m0)
    mvec = jnp.full((16,), mstab, jnp.float32)
    o_part, s0, s1 = _sc(h, a_s.reshape(N), a_d.reshape(N), src, dst, mvec)
    return _fin(o_part, s0.reshape(N, 1), s1.reshape(N, 1), bias.reshape(1, C))
